# baseline (speedup 1.0000x reference)
import functools

import jax
import jax.numpy as jnp
from jax import lax
from jax.experimental import pallas as pl
from jax.experimental.pallas import tpu as pltpu
from jax.experimental.pallas import tpu_sc as plsc

B_, S_, D_, T_ = 4, 2048, 1024, 128
N_ = B_ * S_
LAB_ROWS = N_ // 1024
EPS = 1e-8
INV_LN2 = 1.4426950408889634

NC, NS = 2, 16
NW = NC * NS
NG = NW // 2
DH = D_ // 2
TOK_PER_G = N_ // NG
CH = 32
NCHUNK = TOK_PER_G // CH


def _sc_segsum_body(x_hbm, lab_hbm, zeros_hbm, out_hbm,
                    buf0, buf1, labv, acc, sem0, sem1):
    c = lax.axis_index("c")
    s = lax.axis_index("s")
    wid = c * NS + s
    g = wid // 2
    h = wid % 2
    tok_base = g * TOK_PER_G
    bufs = (buf0, buf1)
    sems = (sem0, sem1)

    def chunk_src(j):
        return x_hbm.at[pl.ds(tok_base + j * CH, CH), pl.ds(h * DH, DH)]

    pltpu.sync_copy(lab_hbm.at[pl.ds(tok_base, TOK_PER_G)], labv)
    pltpu.async_copy(chunk_src(0), buf0, sem0)
    pltpu.async_copy(chunk_src(1), buf1, sem1)
    pltpu.sync_copy(zeros_hbm, acc)

    def chunk_pair(j2, carry):
        for b in range(2):
            j = j2 * 2 + b
            buf = bufs[b]
            pltpu.make_async_copy(chunk_src(j), buf, sems[b]).wait()
            for ii in range(CH // 16):
                lab16 = labv[pl.ds(j * CH + ii * 16, 16)]
                for t in range(16):
                    lab = lab16[t]
                    row = ii * 16 + t
                    for k in range(DH // 16):
                        plsc.addupdate(acc.at[lab, pl.ds(k * 16, 16)],
                                       buf[row, pl.ds(k * 16, 16)])

            @pl.when(j + 2 < NCHUNK)
            def _():
                pltpu.async_copy(chunk_src(j + 2), buf, sems[b])
        return carry

    lax.fori_loop(0, NCHUNK // 2, chunk_pair, 0)
    pltpu.sync_copy(acc, out_hbm.at[wid])


_sc_segsum = functools.partial(
    pl.kernel,
    out_type=jax.ShapeDtypeStruct((NW, T_, DH), jnp.float32),
    mesh=plsc.VectorSubcoreMesh(
        core_axis_name="c", subcore_axis_name="s",
        num_cores=NC, num_subcores=NS),
    scratch_types=[
        pltpu.VMEM((CH, DH), jnp.float32),
        pltpu.VMEM((CH, DH), jnp.float32),
        pltpu.VMEM((TOK_PER_G,), jnp.int32),
        pltpu.VMEM((T_, DH), jnp.float32),
        pltpu.SemaphoreType.DMA,
        pltpu.SemaphoreType.DMA,
    ],
)(_sc_segsum_body)


def _epilogue_body(sp_ref, lab_ref, w_ref, b_ref, proto_ref, temp_ref, out_ref):
    s_h0 = sp_ref[0]
    s_h1 = sp_ref[1]
    for g in range(1, NG):
        s_h0 = s_h0 + sp_ref[2 * g]
        s_h1 = s_h1 + sp_ref[2 * g + 1]
    S = jnp.concatenate([s_h0, s_h1], axis=1)
    temp = temp_ref[0, 0]

    tag_iota = lax.broadcasted_iota(jnp.int32, (T_, 1), 0)
    counts = jnp.zeros((T_, 1), jnp.float32)
    for c in range(LAB_ROWS):
        row = lab_ref[c:c + 1, :]
        oh = (row == tag_iota).astype(jnp.float32)
        counts = counts + jnp.sum(oh, axis=1, keepdims=True)

    sums = lax.dot_general(
        S, w_ref[...], (((1,), (1,)), ((), ())),
        precision=lax.Precision.HIGHEST,
        preferred_element_type=jnp.float32)
    sums = sums + counts * b_ref[...]
    means = sums / jnp.maximum(counts, 1.0)

    mn = means / jnp.maximum(
        jnp.sqrt(jnp.sum(means * means, axis=1, keepdims=True)), EPS)
    proto = proto_ref[...]
    pn = proto / jnp.maximum(
        jnp.sqrt(jnp.sum(proto * proto, axis=1, keepdims=True)), EPS)

    cos_t = lax.dot_general(
        pn, mn, (((1,), (1,)), ((), ())),
        precision=lax.Precision.HIGHEST,
        preferred_element_type=jnp.float32)
    ap_t = -(1.0 - cos_t) / temp

    sim = lax.dot_general(
        pn, pn, (((1,), (1,)), ((), ())),
        precision=lax.Precision.HIGHEST,
        preferred_element_type=jnp.float32)

    BK = 8
    rank_t = jnp.zeros((T_, T_), jnp.float32)
    sim3 = sim[None, :, :]
    jmat = lax.broadcasted_iota(jnp.int32, (BK, T_, T_), 1)
    for kb in range(T_ // BK):
        blk = sim[kb * BK:(kb + 1) * BK, :]
        blk3 = blk[:, None, :]
        kvec = kb * BK + lax.broadcasted_iota(jnp.int32, (BK, T_, T_), 0)
        gt = (blk3 > sim3).astype(jnp.float32)
        eq = jnp.where((blk3 == sim3) & (kvec < jmat), 1.0, 0.0)
        rank_t = rank_t + jnp.sum(gt + eq, axis=0)

    disc_t = jnp.log(rank_t + 2.0) * INV_LN2
    x = ap_t / disc_t

    m = jnp.max(x, axis=0, keepdims=True)
    z = x - m
    lse = jnp.log(jnp.sum(jnp.exp(z), axis=0, keepdims=True))
    logp = z - lse

    eye = (lax.broadcasted_iota(jnp.int32, (T_, T_), 0)
           == lax.broadcasted_iota(jnp.int32, (T_, T_), 1))
    present = counts > 0.0
    val = jnp.where(eye & present, -logp, 0.0)
    total = jnp.sum(jnp.sum(val, axis=1, keepdims=True), axis=0, keepdims=True)
    out_ref[...] = total / jnp.float32(T_)


@jax.jit
def _run(features, labels, W, b, proto, temperature):
    x2 = features.reshape(N_, D_)
    labc = labels.reshape(N_).astype(jnp.int32)
    lab2 = labels.reshape(LAB_ROWS, 1024).astype(jnp.int32)
    b2 = b.reshape(1, T_).astype(jnp.float32)
    temp = jnp.asarray(temperature, jnp.float32).reshape(1, 1)
    zeros = jnp.zeros((T_, DH), jnp.float32)

    S_p = _sc_segsum(x2, labc, zeros)

    loss = pl.pallas_call(
        _epilogue_body,
        out_shape=jax.ShapeDtypeStruct((1, 1), jnp.float32),
    )(S_p, lab2, W.astype(jnp.float32), b2, proto.astype(jnp.float32), temp)
    return loss.reshape(1)


def kernel(features, labels, W, b, proto, temperature=0.3):
    return _run(features, labels, W, b, proto, temperature)


# speedup vs baseline: 1.2416x; 1.2416x over previous
import functools

import jax
import jax.numpy as jnp
from jax import lax
from jax.experimental import pallas as pl
from jax.experimental.pallas import tpu as pltpu
from jax.experimental.pallas import tpu_sc as plsc

B_, S_, D_, T_ = 4, 2048, 1024, 128
N_ = B_ * S_
LAB_ROWS = N_ // 1024
EPS = 1e-8
INV_LN2 = 1.4426950408889634

NC, NS = 2, 16
NW = NC * NS
NG = NW // 2
DH = D_ // 2
TOK_PER_G = N_ // NG
CH = 32
NCHUNK = TOK_PER_G // CH
ZR = 16


def _sc_segsum_body(x_hbm, lab_hbm, out_hbm,
                    buf0, buf1, acc, lab_v, sem0, sem1):
    c = lax.axis_index("c")
    s = lax.axis_index("s")
    wid = c * NS + s
    g = wid // 2
    h = wid % 2
    tok_base = g * TOK_PER_G
    bufs = (buf0, buf1)
    sems = (sem0, sem1)

    def src(j):
        return x_hbm.at[pl.ds(tok_base + j * CH, CH), pl.ds(h * DH, DH)]

    pltpu.sync_copy(lab_hbm.at[g], lab_v)
    pltpu.async_copy(src(0), buf0, sem0)
    pltpu.async_copy(src(1), buf1, sem1)

    zero = jnp.zeros((16,), jnp.float32)

    def zero_body(r, carry):
        for k in range(DH // 16):
            acc[r, pl.ds(k * 16, 16)] = zero
        return carry

    lax.fori_loop(0, T_, zero_body, 0)

    def pair_body(p, carry):
        for b in range(2):
            j = p * 2 + b
            pltpu.make_async_copy(src(j), bufs[b], sems[b]).wait()

            def grp_body(q, c2, _j=j, _buf=bufs[b]):
                base = q * 16
                lab16 = lab_v[pl.ds(_j * CH + base, 16)]
                for t in range(16):
                    lab = lab16[t]
                    for k in range(DH // 16):
                        plsc.addupdate(acc.at[lab, pl.ds(k * 16, 16)],
                                       _buf[base + t, pl.ds(k * 16, 16)])
                return c2

            lax.fori_loop(0, CH // 16, grp_body, 0)

            @pl.when(j + 2 < NCHUNK)
            def _(_j=j, _b=b):
                pltpu.async_copy(src(_j + 2), bufs[_b], sems[_b])
        return carry

    lax.fori_loop(0, NCHUNK // 2, pair_body, 0)

    pltpu.sync_copy(acc, out_hbm.at[wid])


_sc_segsum = functools.partial(
    pl.kernel,
    out_type=jax.ShapeDtypeStruct((NW, T_, DH), jnp.float32),
    mesh=plsc.VectorSubcoreMesh(core_axis_name="c", subcore_axis_name="s"),
    scratch_types=[
        pltpu.VMEM((CH, DH), jnp.float32),
        pltpu.VMEM((CH, DH), jnp.float32),
        pltpu.VMEM((T_, DH), jnp.float32),
        pltpu.VMEM((TOK_PER_G,), jnp.int32),
        pltpu.SemaphoreType.DMA,
        pltpu.SemaphoreType.DMA,
    ],
)(_sc_segsum_body)


def _epilogue_body(sp_ref, lab_ref, w_ref, b_ref, proto_ref, temp_ref, out_ref):
    s_h0 = sp_ref[0]
    s_h1 = sp_ref[1]
    for g in range(1, NG):
        s_h0 = s_h0 + sp_ref[2 * g]
        s_h1 = s_h1 + sp_ref[2 * g + 1]
    S = jnp.concatenate([s_h0, s_h1], axis=1)
    temp = temp_ref[0, 0]

    tag_iota = lax.broadcasted_iota(jnp.int32, (T_, 1), 0)
    counts = jnp.zeros((T_, 1), jnp.float32)
    for c in range(LAB_ROWS):
        row = lab_ref[c:c + 1, :]
        oh = (row == tag_iota).astype(jnp.float32)
        counts = counts + jnp.sum(oh, axis=1, keepdims=True)

    sums = lax.dot_general(
        S, w_ref[...], (((1,), (1,)), ((), ())),
        precision=lax.Precision.HIGHEST,
        preferred_element_type=jnp.float32)
    sums = sums + counts * b_ref[...]
    means = sums / jnp.maximum(counts, 1.0)

    mn = means / jnp.maximum(
        jnp.sqrt(jnp.sum(means * means, axis=1, keepdims=True)), EPS)
    proto = proto_ref[...]
    pn = proto / jnp.maximum(
        jnp.sqrt(jnp.sum(proto * proto, axis=1, keepdims=True)), EPS)

    cos_t = lax.dot_general(
        pn, mn, (((1,), (1,)), ((), ())),
        precision=lax.Precision.HIGHEST,
        preferred_element_type=jnp.float32)
    ap_t = -(1.0 - cos_t) / temp

    sim = lax.dot_general(
        pn, pn, (((1,), (1,)), ((), ())),
        precision=lax.Precision.HIGHEST,
        preferred_element_type=jnp.float32)

    BK = 8
    rank_t = jnp.zeros((T_, T_), jnp.float32)
    sim3 = sim[None, :, :]
    jmat = lax.broadcasted_iota(jnp.int32, (BK, T_, T_), 1)
    for kb in range(T_ // BK):
        blk = sim[kb * BK:(kb + 1) * BK, :]
        blk3 = blk[:, None, :]
        kvec = kb * BK + lax.broadcasted_iota(jnp.int32, (BK, T_, T_), 0)
        gt = (blk3 > sim3).astype(jnp.float32)
        eq = jnp.where((blk3 == sim3) & (kvec < jmat), 1.0, 0.0)
        rank_t = rank_t + jnp.sum(gt + eq, axis=0)

    disc_t = jnp.log(rank_t + 2.0) * INV_LN2
    x = ap_t / disc_t

    m = jnp.max(x, axis=0, keepdims=True)
    z = x - m
    lse = jnp.log(jnp.sum(jnp.exp(z), axis=0, keepdims=True))
    logp = z - lse

    eye = (lax.broadcasted_iota(jnp.int32, (T_, T_), 0)
           == lax.broadcasted_iota(jnp.int32, (T_, T_), 1))
    present = counts > 0.0
    val = jnp.where(eye & present, -logp, 0.0)
    total = jnp.sum(jnp.sum(val, axis=1, keepdims=True), axis=0, keepdims=True)
    out_ref[...] = total / jnp.float32(T_)


@jax.jit
def _run(features, labels, W, b, proto, temperature):
    x2 = features.reshape(N_, D_)
    lab3 = labels.reshape(NG, TOK_PER_G).astype(jnp.int32)
    lab2 = labels.reshape(LAB_ROWS, 1024).astype(jnp.int32)
    b2 = b.reshape(1, T_).astype(jnp.float32)
    temp = jnp.asarray(temperature, jnp.float32).reshape(1, 1)

    S_p = _sc_segsum(x2, lab3)

    loss = pl.pallas_call(
        _epilogue_body,
        out_shape=jax.ShapeDtypeStruct((1, 1), jnp.float32),
    )(S_p, lab2, W.astype(jnp.float32), b2, proto.astype(jnp.float32), temp)
    return loss.reshape(1)


def kernel(features, labels, W, b, proto, temperature=0.3):
    return _run(features, labels, W, b, proto, temperature)


# speedup vs baseline: 2.1803x; 1.7560x over previous
import functools

import jax
import jax.numpy as jnp
from jax import lax
from jax.experimental import pallas as pl
from jax.experimental.pallas import tpu as pltpu
from jax.experimental.pallas import tpu_sc as plsc

B_, S_, D_, T_ = 4, 2048, 1024, 128
N_ = B_ * S_
LAB_ROWS = N_ // 1024
EPS = 1e-8
INV_LN2 = 1.4426950408889634

NC, NS = 2, 16
NW = NC * NS
TOK_PER_W = N_ // NW
CHI = 32
NCHUNK = TOK_PER_W // CHI
STRIPE = T_ // NS


def _sc_segsum_body(x_hbm, lab_hbm, out_hbm,
                    idx_v, zbuf, buf0, buf1, sem0, sem1):
    c = lax.axis_index("c")
    s = lax.axis_index("s")
    wid = c * NS + s
    tok_base = wid * TOK_PER_W
    bufs = (buf0, buf1)
    sems = (sem0, sem1)

    def src(j):
        return x_hbm.at[pl.ds(tok_base + j * CHI, CHI)]

    pltpu.sync_copy(lab_hbm.at[wid], idx_v)
    pltpu.async_copy(src(0), buf0, sem0)
    pltpu.async_copy(src(1), buf1, sem1)

    zero = jnp.zeros((16,), jnp.float32)

    def zrow(r, carry):
        for k in range(D_ // 16):
            zbuf[r, pl.ds(k * 16, 16)] = zero
        return carry

    lax.fori_loop(0, STRIPE, zrow, 0)
    pltpu.sync_copy(zbuf, out_hbm.at[pl.ds(c * T_ + s * STRIPE, STRIPE)])
    plsc.subcore_barrier()

    for j in range(NCHUNK):
        b = j % 2
        pltpu.make_async_copy(src(j), bufs[b], sems[b]).wait()
        pltpu.sync_copy(bufs[b], out_hbm.at[idx_v.at[j]], add=True)
        if j + 2 < NCHUNK:
            pltpu.async_copy(src(j + 2), bufs[b], sems[b])


_sc_segsum = functools.partial(
    pl.kernel,
    out_type=jax.ShapeDtypeStruct((NC * T_, D_), jnp.float32),
    mesh=plsc.VectorSubcoreMesh(core_axis_name="c", subcore_axis_name="s"),
    scratch_types=[
        pltpu.VMEM((NCHUNK, CHI), jnp.int32),
        pltpu.VMEM((STRIPE, D_), jnp.float32),
        pltpu.VMEM((CHI, D_), jnp.float32),
        pltpu.VMEM((CHI, D_), jnp.float32),
        pltpu.SemaphoreType.DMA,
        pltpu.SemaphoreType.DMA,
    ],
)(_sc_segsum_body)


def _epilogue_body(sp_ref, lab_ref, w_ref, b_ref, proto_ref, temp_ref, out_ref):
    S = sp_ref[0] + sp_ref[1]
    temp = temp_ref[0, 0]

    tag_iota = lax.broadcasted_iota(jnp.int32, (T_, 1), 0)
    counts = jnp.zeros((T_, 1), jnp.float32)
    for c in range(LAB_ROWS):
        row = lab_ref[c:c + 1, :]
        oh = (row == tag_iota).astype(jnp.float32)
        counts = counts + jnp.sum(oh, axis=1, keepdims=True)

    sums = lax.dot_general(
        S, w_ref[...], (((1,), (1,)), ((), ())),
        precision=lax.Precision.HIGHEST,
        preferred_element_type=jnp.float32)
    sums = sums + counts * b_ref[...]
    means = sums / jnp.maximum(counts, 1.0)

    mn = means / jnp.maximum(
        jnp.sqrt(jnp.sum(means * means, axis=1, keepdims=True)), EPS)
    proto = proto_ref[...]
    pn = proto / jnp.maximum(
        jnp.sqrt(jnp.sum(proto * proto, axis=1, keepdims=True)), EPS)

    cos_t = lax.dot_general(
        pn, mn, (((1,), (1,)), ((), ())),
        precision=lax.Precision.HIGHEST,
        preferred_element_type=jnp.float32)
    ap_t = -(1.0 - cos_t) / temp

    sim = lax.dot_general(
        pn, pn, (((1,), (1,)), ((), ())),
        precision=lax.Precision.HIGHEST,
        preferred_element_type=jnp.float32)

    BK = 8
    rank_t = jnp.zeros((T_, T_), jnp.float32)
    sim3 = sim[None, :, :]
    jmat = lax.broadcasted_iota(jnp.int32, (BK, T_, T_), 1)
    for kb in range(T_ // BK):
        blk = sim[kb * BK:(kb + 1) * BK, :]
        blk3 = blk[:, None, :]
        kvec = kb * BK + lax.broadcasted_iota(jnp.int32, (BK, T_, T_), 0)
        gt = (blk3 > sim3).astype(jnp.float32)
        eq = jnp.where((blk3 == sim3) & (kvec < jmat), 1.0, 0.0)
        rank_t = rank_t + jnp.sum(gt + eq, axis=0)

    disc_t = jnp.log(rank_t + 2.0) * INV_LN2
    x = ap_t / disc_t

    m = jnp.max(x, axis=0, keepdims=True)
    z = x - m
    lse = jnp.log(jnp.sum(jnp.exp(z), axis=0, keepdims=True))
    logp = z - lse

    eye = (lax.broadcasted_iota(jnp.int32, (T_, T_), 0)
           == lax.broadcasted_iota(jnp.int32, (T_, T_), 1))
    present = counts > 0.0
    val = jnp.where(eye & present, -logp, 0.0)
    total = jnp.sum(jnp.sum(val, axis=1, keepdims=True), axis=0, keepdims=True)
    out_ref[...] = total / jnp.float32(T_)


@jax.jit
def _run(features, labels, W, b, proto, temperature):
    x2 = features.reshape(N_, D_)
    core_off = (jnp.arange(NW, dtype=jnp.int32) // NS * T_)[:, None, None]
    lab3 = labels.reshape(NW, NCHUNK, CHI).astype(jnp.int32) + core_off
    lab2 = labels.reshape(LAB_ROWS, 1024).astype(jnp.int32)
    b2 = b.reshape(1, T_).astype(jnp.float32)
    temp = jnp.asarray(temperature, jnp.float32).reshape(1, 1)

    S_p = _sc_segsum(x2, lab3).reshape(NC, T_, D_)

    loss = pl.pallas_call(
        _epilogue_body,
        out_shape=jax.ShapeDtypeStruct((1, 1), jnp.float32),
    )(S_p, lab2, W.astype(jnp.float32), b2, proto.astype(jnp.float32), temp)
    return loss.reshape(1)


def kernel(features, labels, W, b, proto, temperature=0.3):
    return _run(features, labels, W, b, proto, temperature)


# speedup vs baseline: 3.0578x; 1.4025x over previous
import functools

import jax
import jax.numpy as jnp
from jax import lax
from jax.experimental import pallas as pl
from jax.experimental.pallas import tpu as pltpu
from jax.experimental.pallas import tpu_sc as plsc

B_, S_, D_, T_ = 4, 2048, 1024, 128
N_ = B_ * S_
LAB_ROWS = N_ // 1024
EPS = 1e-8
INV_LN2 = 1.4426950408889634

N_SC = 1024
TOK_TILE = 1024
N_TILES = (N_ - N_SC) // TOK_TILE
SC_TILES = N_SC // TOK_TILE

NC, NS = 2, 16
NW = NC * NS
TOK_PER_W = N_SC // NW
CHI = 32
NCHUNK = TOK_PER_W // CHI
STRIPE = T_ // NS


def _sc_segsum_body(x_hbm, lab_hbm, out_hbm,
                    idx_v, zbuf, buf0, buf1, sem0, sem1):
    c = lax.axis_index("c")
    s = lax.axis_index("s")
    wid = c * NS + s
    tok_base = wid * TOK_PER_W
    bufs = (buf0, buf1)
    sems = (sem0, sem1)

    def src(j):
        return x_hbm.at[pl.ds(tok_base + j * CHI, CHI)]

    pltpu.sync_copy(lab_hbm.at[wid], idx_v)
    pltpu.async_copy(src(0), buf0, sem0)
    if NCHUNK > 1:
        pltpu.async_copy(src(1), buf1, sem1)

    zero = jnp.zeros((16,), jnp.float32)

    def zrow(r, carry):
        for k in range(D_ // 16):
            zbuf[r, pl.ds(k * 16, 16)] = zero
        return carry

    lax.fori_loop(0, STRIPE, zrow, 0)
    pltpu.sync_copy(zbuf, out_hbm.at[pl.ds(c * T_ + s * STRIPE, STRIPE)])
    plsc.subcore_barrier()

    for j in range(NCHUNK):
        b = j % 2
        pltpu.make_async_copy(src(j), bufs[b], sems[b]).wait()
        pltpu.sync_copy(bufs[b], out_hbm.at[idx_v.at[j]], add=True)
        if j + 2 < NCHUNK:
            pltpu.async_copy(src(j + 2), bufs[b], sems[b])


_sc_segsum = functools.partial(
    pl.kernel,
    out_type=jax.ShapeDtypeStruct((NC * T_, D_), jnp.float32),
    mesh=plsc.VectorSubcoreMesh(core_axis_name="c", subcore_axis_name="s"),
    scratch_types=[
        pltpu.VMEM((NCHUNK, CHI), jnp.int32),
        pltpu.VMEM((STRIPE, D_), jnp.float32),
        pltpu.VMEM((CHI, D_), jnp.float32),
        pltpu.VMEM((CHI, D_), jnp.float32),
        pltpu.SemaphoreType.DMA,
        pltpu.SemaphoreType.DMA,
    ],
)(_sc_segsum_body)


def _tc_segsum_body(lab_ref, x_ref, s_ref):
    pid = pl.program_id(0)

    @pl.when(pid == 0)
    def _():
        s_ref[...] = jnp.zeros_like(s_ref)

    lab_row = lab_ref[pl.ds(pid + SC_TILES, 1), :]
    tag_iota = lax.broadcasted_iota(jnp.int32, (T_, 1), 0)
    onehot_t = (lab_row == tag_iota).astype(jnp.float32)
    s_ref[...] += lax.dot_general(
        onehot_t, x_ref[...], (((1,), (0,)), ((), ())),
        preferred_element_type=jnp.float32)


def _epilogue_body(sp_ref, stc_ref, lab_ref, w_ref, b_ref, proto_ref,
                   temp_ref, out_ref):
    S = sp_ref[0] + sp_ref[1] + stc_ref[...]
    temp = temp_ref[0, 0]

    tag_iota = lax.broadcasted_iota(jnp.int32, (T_, 1), 0)
    counts = jnp.zeros((T_, 1), jnp.float32)
    for c in range(LAB_ROWS):
        row = lab_ref[c:c + 1, :]
        oh = (row == tag_iota).astype(jnp.float32)
        counts = counts + jnp.sum(oh, axis=1, keepdims=True)

    sums = lax.dot_general(
        S, w_ref[...], (((1,), (1,)), ((), ())),
        precision=lax.Precision.HIGHEST,
        preferred_element_type=jnp.float32)
    sums = sums + counts * b_ref[...]
    means = sums / jnp.maximum(counts, 1.0)

    mn = means / jnp.maximum(
        jnp.sqrt(jnp.sum(means * means, axis=1, keepdims=True)), EPS)
    proto = proto_ref[...]
    pn = proto / jnp.maximum(
        jnp.sqrt(jnp.sum(proto * proto, axis=1, keepdims=True)), EPS)

    cos_t = lax.dot_general(
        pn, mn, (((1,), (1,)), ((), ())),
        precision=lax.Precision.HIGHEST,
        preferred_element_type=jnp.float32)
    ap_t = -(1.0 - cos_t) / temp

    sim = lax.dot_general(
        pn, pn, (((1,), (1,)), ((), ())),
        precision=lax.Precision.HIGHEST,
        preferred_element_type=jnp.float32)

    BK = 8
    rank_t = jnp.zeros((T_, T_), jnp.float32)
    sim3 = sim[None, :, :]
    jmat = lax.broadcasted_iota(jnp.int32, (BK, T_, T_), 1)
    for kb in range(T_ // BK):
        blk = sim[kb * BK:(kb + 1) * BK, :]
        blk3 = blk[:, None, :]
        kvec = kb * BK + lax.broadcasted_iota(jnp.int32, (BK, T_, T_), 0)
        gt = (blk3 > sim3).astype(jnp.float32)
        eq = jnp.where((blk3 == sim3) & (kvec < jmat), 1.0, 0.0)
        rank_t = rank_t + jnp.sum(gt + eq, axis=0)

    disc_t = jnp.log(rank_t + 2.0) * INV_LN2
    x = ap_t / disc_t

    m = jnp.max(x, axis=0, keepdims=True)
    z = x - m
    lse = jnp.log(jnp.sum(jnp.exp(z), axis=0, keepdims=True))
    logp = z - lse

    eye = (lax.broadcasted_iota(jnp.int32, (T_, T_), 0)
           == lax.broadcasted_iota(jnp.int32, (T_, T_), 1))
    present = counts > 0.0
    val = jnp.where(eye & present, -logp, 0.0)
    total = jnp.sum(jnp.sum(val, axis=1, keepdims=True), axis=0, keepdims=True)
    out_ref[...] = total / jnp.float32(T_)


@jax.jit
def _run(features, labels, W, b, proto, temperature):
    x2 = features.reshape(N_, D_)
    lab1 = labels.reshape(N_).astype(jnp.int32)
    core_off = (jnp.arange(NW, dtype=jnp.int32) // NS * T_)[:, None, None]
    lab3 = lab1[:N_SC].reshape(NW, NCHUNK, CHI) + core_off
    lab2 = lab1.reshape(LAB_ROWS, 1024)
    b2 = b.reshape(1, T_).astype(jnp.float32)
    temp = jnp.asarray(temperature, jnp.float32).reshape(1, 1)

    S_p = _sc_segsum(x2, lab3).reshape(NC, T_, D_)

    S_tc = pl.pallas_call(
        _tc_segsum_body,
        grid=(N_TILES,),
        in_specs=[
            pl.BlockSpec((LAB_ROWS, 1024), lambda g: (0, 0)),
            pl.BlockSpec((TOK_TILE, D_), lambda g: (g + SC_TILES, 0)),
        ],
        out_specs=pl.BlockSpec((T_, D_), lambda g: (0, 0)),
        out_shape=jax.ShapeDtypeStruct((T_, D_), jnp.float32),
    )(lab2.reshape(LAB_ROWS, 1024), x2)

    loss = pl.pallas_call(
        _epilogue_body,
        out_shape=jax.ShapeDtypeStruct((1, 1), jnp.float32),
    )(S_p, S_tc, lab2, W.astype(jnp.float32), b2,
      proto.astype(jnp.float32), temp)
    return loss.reshape(1)


def kernel(features, labels, W, b, proto, temperature=0.3):
    return _run(features, labels, W, b, proto, temperature)
